# baseline (speedup 1.0000x reference)
import functools

import jax
import jax.numpy as jnp
from jax import lax
from jax.experimental import pallas as pl
from jax.experimental.pallas import tpu as pltpu
from jax.experimental.pallas import tpu_sc as plsc


def _sc_gather(emb, y):
    B = y.shape[0]
    _, D = emb.shape
    info = plsc.get_sparse_core_info()
    NC, NS = info.num_cores, info.num_subcores
    NW = NC * NS
    b_per_w = B // NW
    mesh = plsc.VectorSubcoreMesh(core_axis_name="c", subcore_axis_name="s")

    @functools.partial(
        pl.kernel,
        mesh=mesh,
        out_type=jax.ShapeDtypeStruct((B, D), jnp.float32),
        scratch_types=[
            pltpu.VMEM((b_per_w,), jnp.int32),
            pltpu.VMEM((b_per_w, D), jnp.float32),
            pltpu.SemaphoreType.DMA,
        ],
    )
    def gather_kernel(emb_hbm, y_hbm, out_hbm, idx_v, rows_v, sem):
        wid = lax.axis_index("s") * NC + lax.axis_index("c")
        base = wid * b_per_w
        pltpu.sync_copy(y_hbm.at[pl.ds(base, b_per_w)], idx_v)
        pltpu.async_copy(emb_hbm.at[idx_v], rows_v, sem).wait()
        pltpu.sync_copy(rows_v, out_hbm.at[pl.ds(base, b_per_w)])

    return gather_kernel(emb, y)


def _fused_body(y_ref, x_ref, *rest):
    o_ref = rest[-1]
    e_refs = rest[:-1]
    rows = jnp.concatenate([er[...] for er in e_refs], axis=0)
    o_ref[...] = x_ref[...] + rows


def _tc_fused_add(x, y, emb, bb=8):
    B, S, D = x.shape
    emb3 = emb.reshape(emb.shape[0], 1, D)

    def emap(j):
        return lambda i, yr: (yr[i * bb + j], 0, 0)

    return pl.pallas_call(
        _fused_body,
        grid_spec=pltpu.PrefetchScalarGridSpec(
            num_scalar_prefetch=1,
            grid=(B // bb,),
            in_specs=[pl.BlockSpec((bb, S, D), lambda i, yr: (i, 0, 0))]
            + [pl.BlockSpec((1, 1, D), emap(j)) for j in range(bb)],
            out_specs=pl.BlockSpec((bb, S, D), lambda i, yr: (i, 0, 0)),
        ),
        out_shape=jax.ShapeDtypeStruct((B, S, D), x.dtype),
    )(y, x, *([emb3] * bb))


def _add_body(x_ref, e_ref, o_ref):
    o_ref[...] = x_ref[...] + e_ref[...][:, None, :]


def _tc_add(x, e):
    B, S, D = x.shape
    BB = 128
    return pl.pallas_call(
        _add_body,
        grid=(B // BB,),
        in_specs=[
            pl.BlockSpec((BB, S, D), lambda i: (i, 0, 0)),
            pl.BlockSpec((BB, D), lambda i: (i, 0)),
        ],
        out_specs=pl.BlockSpec((BB, S, D), lambda i: (i, 0, 0)),
        out_shape=jax.ShapeDtypeStruct((B, S, D), x.dtype),
    )(x, e)


def _sc_full(x, y, emb):
    B, S, D = x.shape
    info = plsc.get_sparse_core_info()
    NC, NS, L = info.num_cores, info.num_subcores, info.num_lanes
    NW = NC * NS
    bw = B // NW
    NBUF = 3
    mesh = plsc.VectorSubcoreMesh(core_axis_name="c", subcore_axis_name="s")

    @functools.partial(
        pl.kernel,
        mesh=mesh,
        out_type=jax.ShapeDtypeStruct((B, S, D), jnp.float32),
        scratch_types=[
            pltpu.VMEM((bw,), jnp.int32),
            pltpu.VMEM((bw, D), jnp.float32),
            pltpu.SemaphoreType.DMA,
        ]
        + [pltpu.VMEM((S, D), jnp.float32) for _ in range(NBUF)]
        + [pltpu.SemaphoreType.DMA for _ in range(NBUF)]
        + [pltpu.SemaphoreType.DMA for _ in range(NBUF)],
    )
    def body(x_hbm, y_hbm, emb_hbm, out_hbm, idx_v, rows_v, gsem, *bufsems):
        bufs = bufsems[:NBUF]
        isems = bufsems[NBUF:2 * NBUF]
        osems = bufsems[2 * NBUF:]
        wid = lax.axis_index("s") * NC + lax.axis_index("c")
        base = wid * bw
        pltpu.sync_copy(y_hbm.at[pl.ds(base, bw)], idx_v)
        pltpu.async_copy(emb_hbm.at[idx_v], rows_v, gsem).wait()

        in_cp = [None] * NBUF
        out_cp = [None] * NBUF
        for b in range(bw + 1):
            r = b % NBUF
            if b < bw:
                if out_cp[r] is not None:
                    out_cp[r].wait()
                    out_cp[r] = None
                in_cp[r] = pltpu.async_copy(
                    x_hbm.at[base + b], bufs[r], isems[r])
            if b >= 1:
                pb = b - 1
                pr = pb % NBUF
                in_cp[pr].wait()
                ev = [rows_v[pb, pl.ds(j * L, L)] for j in range(D // L)]
                buf = bufs[pr]

                U = 4

                def add_rows(i, _, buf=buf, ev=ev):
                    for u in range(U):
                        s = i * U + u
                        for j in range(D // L):
                            sl = pl.ds(j * L, L)
                            buf[s, sl] = buf[s, sl] + ev[j]
                    return 0

                lax.fori_loop(0, S // U, add_rows, 0)
                out_cp[pr] = pltpu.async_copy(
                    bufs[pr], out_hbm.at[base + pb], osems[pr])
        for r in range(NBUF):
            if out_cp[r] is not None:
                out_cp[r].wait()

    return body(x, y, emb)


def kernel(x, y, emb):
    y = y.astype(jnp.int32)
    return _sc_full(x, y, emb)


# speedup vs baseline: 1.2245x; 1.2245x over previous
import functools

import jax
import jax.numpy as jnp
from jax import lax
from jax.experimental import pallas as pl
from jax.experimental.pallas import tpu as pltpu
from jax.experimental.pallas import tpu_sc as plsc


def _sc_gather_tail(emb, y, H):
    B = y.shape[0]
    _, D = emb.shape
    info = plsc.get_sparse_core_info()
    NC, NS = info.num_cores, info.num_subcores
    NW = NC * NS
    T = B - H
    bt = T // NW
    mesh = plsc.VectorSubcoreMesh(core_axis_name="c", subcore_axis_name="s")

    @functools.partial(
        pl.kernel,
        mesh=mesh,
        out_type=jax.ShapeDtypeStruct((T, D), jnp.float32),
        scratch_types=[
            pltpu.VMEM((bt,), jnp.int32),
            pltpu.VMEM((bt, D), jnp.float32),
            pltpu.SemaphoreType.DMA,
        ],
    )
    def gather_kernel(emb_hbm, y_hbm, out_hbm, idx_v, rows_v, sem):
        wid = lax.axis_index("s") * NC + lax.axis_index("c")
        base = wid * bt
        pltpu.sync_copy(y_hbm.at[pl.ds(H + base, bt)], idx_v)
        pltpu.async_copy(emb_hbm.at[idx_v], rows_v, sem).wait()
        pltpu.sync_copy(rows_v, out_hbm.at[pl.ds(base, bt)])

    return gather_kernel(emb, y)


def _head_body(nsteps, bb, y_sm, x_ref, emb_any, o_ref, ebuf, sem):
    i = pl.program_id(0)

    def issue(step, slot):
        for j in range(bb):
            pltpu.make_async_copy(
                emb_any.at[y_sm[step * bb + j]], ebuf.at[slot, j], sem.at[slot]
            ).start()

    def drain_and_add(step, slot):
        for j in range(bb):
            pltpu.make_async_copy(
                emb_any.at[y_sm[step * bb + j]], ebuf.at[slot, j], sem.at[slot]
            ).wait()
        rows = ebuf[slot]
        o_ref[...] = x_ref[...] + rows[:, None, :]

    even = lax.rem(i, 2) == 0

    @pl.when(i == 0)
    def _():
        issue(0, 0)

    @pl.when(jnp.logical_and(i + 1 < nsteps, even))
    def _():
        issue(i + 1, 1)

    @pl.when(jnp.logical_and(i + 1 < nsteps, jnp.logical_not(even)))
    def _():
        issue(i + 1, 0)

    @pl.when(even)
    def _():
        drain_and_add(i, 0)

    @pl.when(jnp.logical_not(even))
    def _():
        drain_and_add(i, 1)


def _tc_head(x, y, emb, H, bb):
    B, S, D = x.shape
    nsteps = H // bb
    return pl.pallas_call(
        functools.partial(_head_body, nsteps, bb),
        grid_spec=pltpu.PrefetchScalarGridSpec(
            num_scalar_prefetch=1,
            grid=(nsteps,),
            in_specs=[
                pl.BlockSpec((bb, S, D), lambda i, yr: (i, 0, 0)),
                pl.BlockSpec(memory_space=pl.ANY),
            ],
            out_specs=pl.BlockSpec((bb, S, D), lambda i, yr: (i, 0, 0)),
            scratch_shapes=[
                pltpu.VMEM((2, bb, D), jnp.float32),
                pltpu.SemaphoreType.DMA((2,)),
            ],
        ),
        out_shape=jax.ShapeDtypeStruct((B, S, D), x.dtype),
    )(y, x, emb)


def _tail_body(x_ref, e_ref, _prev_ref, o_ref):
    o_ref[...] = x_ref[...] + e_ref[...][:, None, :]


def _tc_tail(x, e_tail, prev, H, bb):
    B, S, D = x.shape
    off = H // bb
    nsteps = (B - H) // bb
    return pl.pallas_call(
        _tail_body,
        grid=(nsteps,),
        in_specs=[
            pl.BlockSpec((bb, S, D), lambda i: (i + off, 0, 0)),
            pl.BlockSpec((bb, D), lambda i: (i, 0)),
            pl.BlockSpec(memory_space=pl.ANY),
        ],
        out_specs=pl.BlockSpec((bb, S, D), lambda i: (i + off, 0, 0)),
        out_shape=jax.ShapeDtypeStruct((B, S, D), x.dtype),
        input_output_aliases={2: 0},
    )(x, e_tail, prev)


def kernel(x, y, emb):
    y = y.astype(jnp.int32)
    H = 256
    e_tail = _sc_gather_tail(emb, y, H)
    out_head = _tc_head(x, y, emb, H, bb=64)
    return _tc_tail(x, e_tail, out_head, H, bb=128)
